# Initial kernel scaffold; baseline (speedup 1.0000x reference)
import jax
import jax.numpy as jnp
from jax.experimental import pallas as pl


def kernel(xyz, features, new_xyz, W1, b1, W2, b2):
    raise NotImplementedError("write your pallas kernel here")


# speedup vs baseline: 9.7086x; 9.7086x over previous
import functools

import jax
import jax.numpy as jnp
from jax import lax
from jax.experimental import pallas as pl
from jax.experimental.pallas import tpu as pltpu
from jax.experimental.pallas import tpu_sc as plsc

_RADIUS = 0.1
_NSAMPLE = 32

_NCORES = 2
_NSUB = 16
_NWORKERS = _NCORES * _NSUB


def _sc_ball_query_group(xyz_rows, cent_rows, feats_rows, B, N, P, CP):
    NS = _NSAMPLE
    r2 = jnp.float32(_RADIUS * _RADIUS)
    n_steps = N // 16
    workers_per_batch = _NWORKERS // B
    pc = P // workers_per_batch

    mesh = plsc.VectorSubcoreMesh(core_axis_name="c", subcore_axis_name="s")

    @functools.partial(
        pl.kernel,
        out_type=[
            jax.ShapeDtypeStruct((B * P, NS, CP), jnp.float32),
            jax.ShapeDtypeStruct((B * P, NS * 4), jnp.float32),
            jax.ShapeDtypeStruct((B * P,), jnp.float32),
        ],
        mesh=mesh,
        scratch_types=[
            pltpu.VMEM((N,), jnp.float32),
            pltpu.VMEM((N,), jnp.float32),
            pltpu.VMEM((N,), jnp.float32),
            pltpu.VMEM((P,), jnp.float32),
            pltpu.VMEM((P,), jnp.float32),
            pltpu.VMEM((P,), jnp.float32),
            pltpu.VMEM((NS + 16,), jnp.int32),
            pltpu.VMEM((NS,), jnp.int32),
            pltpu.VMEM((NS, CP), jnp.float32),
            pltpu.VMEM((NS * 4,), jnp.float32),
            pltpu.VMEM((pc,), jnp.float32),
            pltpu.SemaphoreType.DMA,
        ],
        compiler_params=pltpu.CompilerParams(needs_layout_passes=False),
    )
    def sc_kernel(xyz_hbm, cent_hbm, feats_hbm,
                  gfeats_hbm, gxyz_hbm, valid_hbm,
                  xs_v, ys_v, zs_v, cx_v, cy_v, cz_v,
                  cand_v, rowid_v, rows_v, gxyz_v, valid_v, sem):
        wid = lax.axis_index("c") * _NSUB + lax.axis_index("s")
        b = wid // workers_per_batch
        p0 = (wid % workers_per_batch) * pc

        pltpu.sync_copy(xyz_hbm.at[b * 3 + 0], xs_v)
        pltpu.sync_copy(xyz_hbm.at[b * 3 + 1], ys_v)
        pltpu.sync_copy(xyz_hbm.at[b * 3 + 2], zs_v)
        pltpu.sync_copy(cent_hbm.at[b * 3 + 0], cx_v)
        pltpu.sync_copy(cent_hbm.at[b * 3 + 1], cy_v)
        pltpu.sync_copy(cent_hbm.at[b * 3 + 2], cz_v)

        lane = lax.iota(jnp.int32, 16)
        zeros16 = jnp.zeros((16,), jnp.int32)

        for q in range(NS * 4 // 16):
            gxyz_v[pl.ds(q * 16, 16)] = jnp.zeros((16,), jnp.float32)

        def per_center(j, _):
            p = p0 + j
            pv = jnp.full((16,), p, jnp.int32)
            cxv = plsc.load_gather(cx_v, [pv])
            cyv = plsc.load_gather(cy_v, [pv])
            czv = plsc.load_gather(cz_v, [pv])

            def scan_cond(carry):
                i, cnt = carry
                return jnp.logical_and(i < n_steps, cnt < NS)

            def scan_body(carry):
                i, cnt = carry
                base = i * 16
                dx = xs_v[pl.ds(base, 16)] - cxv
                dy = ys_v[pl.ds(base, 16)] - cyv
                dz = zs_v[pl.ds(base, 16)] - czv
                d2 = dx * dx + dy * dy + dz * dz
                mask = d2 < r2
                plsc.store_compressed(cand_v.at[pl.ds(cnt, 16)],
                                      base + lane, mask=mask)
                pops = plsc.all_reduce_population_count(mask)
                return i + 1, cnt + jnp.max(pops)

            _, cnt = lax.while_loop(scan_cond, scan_body,
                                    (jnp.int32(0), jnp.int32(0)))

            has_any = cnt > 0
            c0 = plsc.load_gather(cand_v, [zeros16])
            c0 = jnp.where(has_any, c0, zeros16)
            cntv = jnp.full((16,), cnt, jnp.int32)
            lo = cand_v[pl.ds(0, 16)]
            hi = cand_v[pl.ds(16, 16)]
            idx0 = jnp.where(lane < cntv, lo, c0)
            idx1 = jnp.where(lane + 16 < cntv, hi, c0)

            for c, coords in ((0, xs_v), (1, ys_v), (2, zs_v)):
                cc = (cxv, cyv, czv)[c]
                g0 = plsc.load_gather(coords, [idx0]) - cc
                g1 = plsc.load_gather(coords, [idx1]) - cc
                plsc.store_scatter(gxyz_v, [lane * 4 + c], g0)
                plsc.store_scatter(gxyz_v, [(lane + 16) * 4 + c], g1)

            rowid_v[pl.ds(0, 16)] = idx0 + b * N
            rowid_v[pl.ds(16, 16)] = idx1 + b * N
            pltpu.async_copy(feats_hbm.at[rowid_v], rows_v, sem).wait()

            cg = b * P + p
            pltpu.sync_copy(rows_v, gfeats_hbm.at[cg])
            pltpu.sync_copy(gxyz_v, gxyz_hbm.at[cg])

            vflag = jnp.where(has_any, jnp.float32(1.0), jnp.float32(0.0))
            plsc.store_scatter(valid_v, [jnp.full((16,), j, jnp.int32)],
                               jnp.full((16,), vflag, jnp.float32),
                               mask=lane == 0)
            return 0

        lax.fori_loop(0, pc, per_center, 0)
        pltpu.sync_copy(valid_v, valid_hbm.at[pl.ds((b * P + p0), pc)])

    return sc_kernel(xyz_rows, cent_rows, feats_rows)


def _tc_mlp_maxpool(gfeats, gxyz, valid, w1f, w1x, b1, w2t, b2, BP, CP):
    NS = _NSAMPLE
    BC = 128

    def body(gf_ref, gx_ref, v_ref, w1f_ref, w1x_ref, b1_ref, w2_ref, b2_ref,
             out_ref):
        gf = gf_ref[...].reshape(BC * NS, CP)
        gx = gx_ref[...].reshape(BC * NS, 4)
        h1 = jnp.dot(gf, w1f_ref[...], preferred_element_type=jnp.float32)
        h1 = h1 + jnp.dot(gx, w1x_ref[...], preferred_element_type=jnp.float32)
        h1 = jnp.maximum(h1 + b1_ref[...], 0.0)
        h2 = jnp.dot(h1, w2_ref[...], preferred_element_type=jnp.float32)
        h2 = jnp.maximum(h2 + b2_ref[...], 0.0)
        h2 = h2.reshape(BC, NS, 128)
        out_ref[...] = jnp.max(h2, axis=1) * v_ref[...]

    return pl.pallas_call(
        body,
        grid=(BP // BC,),
        in_specs=[
            pl.BlockSpec((BC, NS, CP), lambda i: (i, 0, 0)),
            pl.BlockSpec((BC, NS, 4), lambda i: (i, 0, 0)),
            pl.BlockSpec((BC, 1), lambda i: (i, 0)),
            pl.BlockSpec((CP, 64), lambda i: (0, 0)),
            pl.BlockSpec((4, 64), lambda i: (0, 0)),
            pl.BlockSpec((1, 64), lambda i: (0, 0)),
            pl.BlockSpec((64, 128), lambda i: (0, 0)),
            pl.BlockSpec((1, 128), lambda i: (0, 0)),
        ],
        out_specs=pl.BlockSpec((BC, 128), lambda i: (i, 0)),
        out_shape=jax.ShapeDtypeStruct((BP, 128), jnp.float32),
    )(gfeats, gxyz, valid, w1f, w1x, b1, w2t, b2)


def kernel(xyz, features, new_xyz, W1, b1, W2, b2):
    B, N, _ = xyz.shape
    P = new_xyz.shape[1]
    C = features.shape[1]

    CP = 128
    xyz_rows = xyz.transpose(0, 2, 1).reshape(B * 3, N)
    cent_rows = new_xyz.transpose(0, 2, 1).reshape(B * 3, P)
    feats_rows = jnp.pad(features.transpose(0, 2, 1).reshape(B * N, C),
                         ((0, 0), (0, CP - C)))

    gfeats, gxyz, valid = _sc_ball_query_group(
        xyz_rows, cent_rows, feats_rows, B, N, P, CP)

    w1f = jnp.pad(W1[:, 3:].T, ((0, CP - C), (0, 0)))
    w1x = jnp.pad(W1[:, :3], ((0, 0), (0, 1))).T
    b1r = b1.reshape(1, 64)
    w2t = W2.T
    b2r = b2.reshape(1, 128)

    out = _tc_mlp_maxpool(gfeats, gxyz.reshape(B * P, _NSAMPLE, 4),
                          valid.reshape(B * P, 1),
                          w1f, w1x, b1r, w2t, b2r, B * P, CP)

    new_features = out.reshape(B, P, 128).transpose(0, 2, 1)
    return new_xyz, new_features


# speedup vs baseline: 38.9835x; 4.0153x over previous
import functools

import jax
import jax.numpy as jnp
from jax import lax
from jax.experimental import pallas as pl
from jax.experimental.pallas import tpu as pltpu
from jax.experimental.pallas import tpu_sc as plsc

_RADIUS = 0.1
_NSAMPLE = 32

_NCORES = 2
_NSUB = 16
_NWORKERS = _NCORES * _NSUB


def _sc_ball_query_group(xyz_rows, cent_rows, feats_rows, B, N, P, CP):
    NS = _NSAMPLE
    r2 = jnp.float32(_RADIUS * _RADIUS)
    BLK = 8
    n_blocks = N // (16 * BLK)
    workers_per_batch = _NWORKERS // B
    pc = P // workers_per_batch
    GC = 4
    n_groups = pc // GC

    mesh = plsc.VectorSubcoreMesh(core_axis_name="c", subcore_axis_name="s")

    @functools.partial(
        pl.kernel,
        out_type=[
            jax.ShapeDtypeStruct((B * P * NS, CP), jnp.float32),
            jax.ShapeDtypeStruct((B * P * NS * 4,), jnp.float32),
            jax.ShapeDtypeStruct((B * P,), jnp.float32),
        ],
        mesh=mesh,
        scratch_types=[
            pltpu.VMEM((N,), jnp.float32),
            pltpu.VMEM((N,), jnp.float32),
            pltpu.VMEM((N,), jnp.float32),
            pltpu.VMEM((P,), jnp.float32),
            pltpu.VMEM((P,), jnp.float32),
            pltpu.VMEM((P,), jnp.float32),
            pltpu.VMEM((176,), jnp.int32),
            [pltpu.VMEM((GC * NS,), jnp.int32)] * 2,
            [pltpu.VMEM((GC * NS, CP), jnp.float32)] * 2,
            pltpu.VMEM((pc * NS * 4,), jnp.float32),
            pltpu.VMEM((pc,), jnp.float32),
            [pltpu.SemaphoreType.DMA] * 2,
            [pltpu.SemaphoreType.DMA] * 2,
        ],
        compiler_params=pltpu.CompilerParams(needs_layout_passes=False),
    )
    def sc_kernel(xyz_hbm, cent_hbm, feats_hbm,
                  gfeats_hbm, gxyz_hbm, valid_hbm,
                  xs_v, ys_v, zs_v, cx_v, cy_v, cz_v,
                  cand_v, idxb, fb, gxyz_v, valid_v, sem_g, sem_o):
        wid = lax.axis_index("c") * _NSUB + lax.axis_index("s")
        b = wid // workers_per_batch
        p0 = (wid % workers_per_batch) * pc
        slab = b * P + p0

        pltpu.sync_copy(xyz_hbm.at[b * 3 + 0], xs_v)
        pltpu.sync_copy(xyz_hbm.at[b * 3 + 1], ys_v)
        pltpu.sync_copy(xyz_hbm.at[b * 3 + 2], zs_v)
        pltpu.sync_copy(cent_hbm.at[b * 3 + 0], cx_v)
        pltpu.sync_copy(cent_hbm.at[b * 3 + 1], cy_v)
        pltpu.sync_copy(cent_hbm.at[b * 3 + 2], cz_v)

        lane = lax.iota(jnp.int32, 16)
        zeros16 = jnp.zeros((16,), jnp.int32)
        zerosf = jnp.zeros((16,), jnp.float32)

        def scan_center(j):
            p = p0 + j
            pv = jnp.full((16,), p, jnp.int32)
            cxv = plsc.load_gather(cx_v, [pv])
            cyv = plsc.load_gather(cy_v, [pv])
            czv = plsc.load_gather(cz_v, [pv])

            def scan_cond(carry):
                i, cnt = carry
                return jnp.logical_and(i < n_blocks, cnt < NS)

            def scan_body(carry):
                i, cnt = carry
                base0 = i * (16 * BLK)
                masks, pops = [], []
                for k in range(BLK):
                    base = base0 + k * 16
                    dx = xs_v[pl.ds(base, 16)] - cxv
                    dy = ys_v[pl.ds(base, 16)] - cyv
                    dz = zs_v[pl.ds(base, 16)] - czv
                    d2 = dx * dx + dy * dy + dz * dz
                    masks.append(d2 < r2)
                    pops.append(plsc.all_reduce_population_count(masks[k]))
                c = cnt
                for k in range(BLK):
                    plsc.store_compressed(cand_v.at[pl.ds(c, 16)],
                                          (base0 + k * 16) + lane,
                                          mask=masks[k])
                    c = c + jnp.max(pops[k])
                return i + 1, c

            _, cnt = lax.while_loop(scan_cond, scan_body,
                                    (jnp.int32(0), jnp.int32(0)))

            has_any = cnt > 0
            c0 = plsc.load_gather(cand_v, [zeros16])
            c0 = jnp.where(has_any, c0, zeros16)
            cntv = jnp.full((16,), cnt, jnp.int32)
            idx0 = jnp.where(lane < cntv, cand_v[pl.ds(0, 16)], c0)
            idx1 = jnp.where(lane + 16 < cntv, cand_v[pl.ds(16, 16)], c0)

            jrow = jnp.full((16,), j * (NS * 4), jnp.int32)
            plsc.store_scatter(gxyz_v, [jrow + lane * 4 + 3], zerosf)
            plsc.store_scatter(gxyz_v, [jrow + (lane + 16) * 4 + 3], zerosf)
            for c, coords in ((0, xs_v), (1, ys_v), (2, zs_v)):
                cc = (cxv, cyv, czv)[c]
                g0 = plsc.load_gather(coords, [idx0]) - cc
                g1 = plsc.load_gather(coords, [idx1]) - cc
                plsc.store_scatter(gxyz_v, [jrow + lane * 4 + c], g0)
                plsc.store_scatter(gxyz_v, [jrow + (lane + 16) * 4 + c], g1)

            vflag = jnp.where(has_any, jnp.float32(1.0), jnp.float32(0.0))
            plsc.store_scatter(valid_v, [jnp.full((16,), j, jnp.int32)],
                               jnp.full((16,), vflag, jnp.float32),
                               mask=lane == 0)
            return idx0, idx1

        def gather_start(g, k):
            pltpu.async_copy(feats_hbm.at[idxb[k]], fb[k], sem_g[k])

        def gather_wait(k):
            pltpu.make_async_copy(feats_hbm.at[idxb[k]], fb[k],
                                  sem_g[k]).wait()

        def out_start(g, k):
            dst = gfeats_hbm.at[pl.ds((slab + g * GC) * NS, GC * NS)]
            pltpu.async_copy(fb[k], dst, sem_o[k])

        def out_wait(g, k):
            dst = gfeats_hbm.at[pl.ds((slab + g * GC) * NS, GC * NS)]
            pltpu.make_async_copy(fb[k], dst, sem_o[k]).wait()

        def pair_body(t, _):
            for k in range(2):
                g = t * 2 + k
                for cloc in range(GC):
                    idx0, idx1 = scan_center(g * GC + cloc)
                    idxb[k][pl.ds(cloc * NS, 16)] = idx0 + b * N
                    idxb[k][pl.ds(cloc * NS + 16, 16)] = idx1 + b * N
                @pl.when(t > 0)
                def _():
                    out_wait(g - 2, k)
                gather_start(g, k)
                @pl.when(g > 0)
                def _():
                    gather_wait(k ^ 1)
                    out_start(g - 1, k ^ 1)
            return 0

        lax.fori_loop(0, n_groups // 2, pair_body, 0)
        gather_wait(1)
        out_start(n_groups - 1, 1)
        out_wait(n_groups - 2, 0)
        out_wait(n_groups - 1, 1)

        pltpu.sync_copy(gxyz_v, gxyz_hbm.at[pl.ds(slab * NS * 4, pc * NS * 4)])
        pltpu.sync_copy(valid_v, valid_hbm.at[pl.ds(slab, pc)])

    return sc_kernel(xyz_rows, cent_rows, feats_rows)


def _tc_mlp_maxpool(gfeats, gxyz, valid, w1f, w1x, b1, w2t, b2, BP, CP):
    NS = _NSAMPLE
    BC = 128

    def body(gf_ref, gx_ref, v_ref, w1f_ref, w1x_ref, b1_ref, w2_ref, b2_ref,
             out_ref):
        gf = gf_ref[...].reshape(BC * NS, CP)
        gx = gx_ref[...].reshape(BC * NS, 4)
        h1 = jnp.dot(gf, w1f_ref[...], preferred_element_type=jnp.float32)
        h1 = h1 + jnp.dot(gx, w1x_ref[...], preferred_element_type=jnp.float32)
        h1 = jnp.maximum(h1 + b1_ref[...], 0.0)
        h2 = jnp.dot(h1, w2_ref[...], preferred_element_type=jnp.float32)
        h2 = jnp.maximum(h2 + b2_ref[...], 0.0)
        h2 = h2.reshape(BC, NS, 128)
        out_ref[...] = jnp.max(h2, axis=1) * v_ref[...]

    return pl.pallas_call(
        body,
        grid=(BP // BC,),
        in_specs=[
            pl.BlockSpec((BC, NS, CP), lambda i: (i, 0, 0)),
            pl.BlockSpec((BC, NS, 4), lambda i: (i, 0, 0)),
            pl.BlockSpec((BC, 1), lambda i: (i, 0)),
            pl.BlockSpec((CP, 64), lambda i: (0, 0)),
            pl.BlockSpec((4, 64), lambda i: (0, 0)),
            pl.BlockSpec((1, 64), lambda i: (0, 0)),
            pl.BlockSpec((64, 128), lambda i: (0, 0)),
            pl.BlockSpec((1, 128), lambda i: (0, 0)),
        ],
        out_specs=pl.BlockSpec((BC, 128), lambda i: (i, 0)),
        out_shape=jax.ShapeDtypeStruct((BP, 128), jnp.float32),
    )(gfeats, gxyz, valid, w1f, w1x, b1, w2t, b2)


def kernel(xyz, features, new_xyz, W1, b1, W2, b2):
    B, N, _ = xyz.shape
    P = new_xyz.shape[1]
    C = features.shape[1]

    CP = 128
    xyz_rows = xyz.transpose(0, 2, 1).reshape(B * 3, N)
    cent_rows = new_xyz.transpose(0, 2, 1).reshape(B * 3, P)
    feats_rows = jnp.pad(features.transpose(0, 2, 1).reshape(B * N, C),
                         ((0, 0), (0, CP - C)))

    gfeats, gxyz, valid = _sc_ball_query_group(
        xyz_rows, cent_rows, feats_rows, B, N, P, CP)

    w1f = jnp.pad(W1[:, 3:].T, ((0, CP - C), (0, 0)))
    w1x = jnp.pad(W1[:, :3], ((0, 0), (0, 1))).T
    b1r = b1.reshape(1, 64)
    w2t = W2.T
    b2r = b2.reshape(1, 128)

    out = _tc_mlp_maxpool(gfeats.reshape(B * P, _NSAMPLE, CP),
                          gxyz.reshape(B * P, _NSAMPLE, 4),
                          valid.reshape(B * P, 1),
                          w1f, w1x, b1r, w2t, b2r, B * P, CP)

    new_features = out.reshape(B, P, 128).transpose(0, 2, 1)
    return new_xyz, new_features


# speedup vs baseline: 38.9935x; 1.0003x over previous
import functools

import jax
import jax.numpy as jnp
from jax import lax
from jax.experimental import pallas as pl
from jax.experimental.pallas import tpu as pltpu
from jax.experimental.pallas import tpu_sc as plsc

_RADIUS = 0.1
_NSAMPLE = 32

_NCORES = 2
_NSUB = 16
_NWORKERS = _NCORES * _NSUB


def _sc_ball_query_group(xyz_rows, cent_rows, feats_rows, B, N, P, CP):
    NS = _NSAMPLE
    r2 = jnp.float32(_RADIUS * _RADIUS)
    BLK = 8
    n_blocks = N // (16 * BLK)
    workers_per_batch = _NWORKERS // B
    pc = P // workers_per_batch
    GC = 4
    n_groups = pc // GC

    mesh = plsc.VectorSubcoreMesh(core_axis_name="c", subcore_axis_name="s")

    @functools.partial(
        pl.kernel,
        out_type=[
            jax.ShapeDtypeStruct((B * P * NS, CP), jnp.float32),
            jax.ShapeDtypeStruct((B * P * NS * 4,), jnp.float32),
            jax.ShapeDtypeStruct((B * P,), jnp.float32),
        ],
        mesh=mesh,
        scratch_types=[
            pltpu.VMEM((N,), jnp.float32),
            pltpu.VMEM((N,), jnp.float32),
            pltpu.VMEM((N,), jnp.float32),
            pltpu.VMEM((P,), jnp.float32),
            pltpu.VMEM((P,), jnp.float32),
            pltpu.VMEM((P,), jnp.float32),
            pltpu.VMEM((176,), jnp.int32),
            [pltpu.VMEM((GC * NS,), jnp.int32)] * 2,
            [pltpu.VMEM((GC * NS, CP), jnp.float32)] * 2,
            pltpu.VMEM((pc * NS * 4,), jnp.float32),
            pltpu.VMEM((pc,), jnp.float32),
            [pltpu.SemaphoreType.DMA] * 2,
            [pltpu.SemaphoreType.DMA] * 2,
        ],
        compiler_params=pltpu.CompilerParams(needs_layout_passes=False),
    )
    def sc_kernel(xyz_hbm, cent_hbm, feats_hbm,
                  gfeats_hbm, gxyz_hbm, valid_hbm,
                  xs_v, ys_v, zs_v, cx_v, cy_v, cz_v,
                  cand_v, idxb, fb, gxyz_v, valid_v, sem_g, sem_o):
        wid = lax.axis_index("c") * _NSUB + lax.axis_index("s")
        b = wid // workers_per_batch
        p0 = (wid % workers_per_batch) * pc
        slab = b * P + p0

        pltpu.sync_copy(xyz_hbm.at[b * 3 + 0], xs_v)
        pltpu.sync_copy(xyz_hbm.at[b * 3 + 1], ys_v)
        pltpu.sync_copy(xyz_hbm.at[b * 3 + 2], zs_v)
        pltpu.sync_copy(cent_hbm.at[b * 3 + 0], cx_v)
        pltpu.sync_copy(cent_hbm.at[b * 3 + 1], cy_v)
        pltpu.sync_copy(cent_hbm.at[b * 3 + 2], cz_v)

        lane = lax.iota(jnp.int32, 16)
        zeros16 = jnp.zeros((16,), jnp.int32)
        zerosf = jnp.zeros((16,), jnp.float32)

        def scan_center(j):
            p = p0 + j
            pv = jnp.full((16,), p, jnp.int32)
            cxv = plsc.load_gather(cx_v, [pv])
            cyv = plsc.load_gather(cy_v, [pv])
            czv = plsc.load_gather(cz_v, [pv])

            def scan_cond(carry):
                i, cnt = carry
                return jnp.logical_and(i < n_blocks, cnt < NS)

            def scan_body(carry):
                i, cnt = carry
                base0 = i * (16 * BLK)
                masks, pops = [], []
                for k in range(BLK):
                    base = base0 + k * 16
                    dx = xs_v[pl.ds(base, 16)] - cxv
                    dy = ys_v[pl.ds(base, 16)] - cyv
                    dz = zs_v[pl.ds(base, 16)] - czv
                    d2 = dx * dx + dy * dy + dz * dz
                    masks.append(d2 < r2)
                    pops.append(plsc.all_reduce_population_count(masks[k]))
                sums = [jnp.max(pops[k]) for k in range(BLK)]
                c = cnt
                for k in range(BLK):
                    plsc.store_compressed(cand_v.at[pl.ds(c, 16)],
                                          (base0 + k * 16) + lane,
                                          mask=masks[k])
                    c = c + sums[k]
                return i + 1, c

            _, cnt = lax.while_loop(scan_cond, scan_body,
                                    (jnp.int32(0), jnp.int32(0)))

            has_any = cnt > 0
            c0 = plsc.load_gather(cand_v, [zeros16])
            c0 = jnp.where(has_any, c0, zeros16)
            cntv = jnp.full((16,), cnt, jnp.int32)
            idx0 = jnp.where(lane < cntv, cand_v[pl.ds(0, 16)], c0)
            idx1 = jnp.where(lane + 16 < cntv, cand_v[pl.ds(16, 16)], c0)

            jrow = jnp.full((16,), j * (NS * 4), jnp.int32)
            plsc.store_scatter(gxyz_v, [jrow + lane * 4 + 3], zerosf)
            plsc.store_scatter(gxyz_v, [jrow + (lane + 16) * 4 + 3], zerosf)
            for c, coords in ((0, xs_v), (1, ys_v), (2, zs_v)):
                cc = (cxv, cyv, czv)[c]
                g0 = plsc.load_gather(coords, [idx0]) - cc
                g1 = plsc.load_gather(coords, [idx1]) - cc
                plsc.store_scatter(gxyz_v, [jrow + lane * 4 + c], g0)
                plsc.store_scatter(gxyz_v, [jrow + (lane + 16) * 4 + c], g1)

            vflag = jnp.where(has_any, jnp.float32(1.0), jnp.float32(0.0))
            plsc.store_scatter(valid_v, [jnp.full((16,), j, jnp.int32)],
                               jnp.full((16,), vflag, jnp.float32),
                               mask=lane == 0)
            return idx0, idx1

        def gather_start(g, k):
            pltpu.async_copy(feats_hbm.at[idxb[k]], fb[k], sem_g[k])

        def gather_wait(k):
            pltpu.make_async_copy(feats_hbm.at[idxb[k]], fb[k],
                                  sem_g[k]).wait()

        def out_start(g, k):
            dst = gfeats_hbm.at[pl.ds((slab + g * GC) * NS, GC * NS)]
            pltpu.async_copy(fb[k], dst, sem_o[k])

        def out_wait(g, k):
            dst = gfeats_hbm.at[pl.ds((slab + g * GC) * NS, GC * NS)]
            pltpu.make_async_copy(fb[k], dst, sem_o[k]).wait()

        def pair_body(t, _):
            for k in range(2):
                g = t * 2 + k
                for cloc in range(GC):
                    idx0, idx1 = scan_center(g * GC + cloc)
                    idxb[k][pl.ds(cloc * NS, 16)] = idx0 + b * N
                    idxb[k][pl.ds(cloc * NS + 16, 16)] = idx1 + b * N
                @pl.when(t > 0)
                def _():
                    out_wait(g - 2, k)
                gather_start(g, k)
                @pl.when(g > 0)
                def _():
                    gather_wait(k ^ 1)
                    out_start(g - 1, k ^ 1)
            return 0

        lax.fori_loop(0, n_groups // 2, pair_body, 0)
        gather_wait(1)
        out_start(n_groups - 1, 1)
        out_wait(n_groups - 2, 0)
        out_wait(n_groups - 1, 1)

        pltpu.sync_copy(gxyz_v, gxyz_hbm.at[pl.ds(slab * NS * 4, pc * NS * 4)])
        pltpu.sync_copy(valid_v, valid_hbm.at[pl.ds(slab, pc)])

    return sc_kernel(xyz_rows, cent_rows, feats_rows)


def _tc_mlp_maxpool(gfeats, gxyz, valid, w1f, w1x, b1, w2t, b2, BP, CP):
    NS = _NSAMPLE
    BC = 128

    def body(gf_ref, gx_ref, v_ref, w1f_ref, w1x_ref, b1_ref, w2_ref, b2_ref,
             out_ref):
        gf = gf_ref[...].reshape(BC * NS, CP)
        gx = gx_ref[...].reshape(BC * NS, 4)
        h1 = jnp.dot(gf, w1f_ref[...], preferred_element_type=jnp.float32)
        h1 = h1 + jnp.dot(gx, w1x_ref[...], preferred_element_type=jnp.float32)
        h1 = jnp.maximum(h1 + b1_ref[...], 0.0)
        h2 = jnp.dot(h1, w2_ref[...], preferred_element_type=jnp.float32)
        h2 = jnp.maximum(h2 + b2_ref[...], 0.0)
        h2 = h2.reshape(BC, NS, 128)
        out_ref[...] = jnp.max(h2, axis=1) * v_ref[...]

    return pl.pallas_call(
        body,
        grid=(BP // BC,),
        in_specs=[
            pl.BlockSpec((BC, NS, CP), lambda i: (i, 0, 0)),
            pl.BlockSpec((BC, NS, 4), lambda i: (i, 0, 0)),
            pl.BlockSpec((BC, 1), lambda i: (i, 0)),
            pl.BlockSpec((CP, 64), lambda i: (0, 0)),
            pl.BlockSpec((4, 64), lambda i: (0, 0)),
            pl.BlockSpec((1, 64), lambda i: (0, 0)),
            pl.BlockSpec((64, 128), lambda i: (0, 0)),
            pl.BlockSpec((1, 128), lambda i: (0, 0)),
        ],
        out_specs=pl.BlockSpec((BC, 128), lambda i: (i, 0)),
        out_shape=jax.ShapeDtypeStruct((BP, 128), jnp.float32),
    )(gfeats, gxyz, valid, w1f, w1x, b1, w2t, b2)


def kernel(xyz, features, new_xyz, W1, b1, W2, b2):
    B, N, _ = xyz.shape
    P = new_xyz.shape[1]
    C = features.shape[1]

    CP = 128
    xyz_rows = xyz.transpose(0, 2, 1).reshape(B * 3, N)
    cent_rows = new_xyz.transpose(0, 2, 1).reshape(B * 3, P)
    feats_rows = jnp.pad(features.transpose(0, 2, 1).reshape(B * N, C),
                         ((0, 0), (0, CP - C)))

    gfeats, gxyz, valid = _sc_ball_query_group(
        xyz_rows, cent_rows, feats_rows, B, N, P, CP)

    w1f = jnp.pad(W1[:, 3:].T, ((0, CP - C), (0, 0)))
    w1x = jnp.pad(W1[:, :3], ((0, 0), (0, 1))).T
    b1r = b1.reshape(1, 64)
    w2t = W2.T
    b2r = b2.reshape(1, 128)

    out = _tc_mlp_maxpool(gfeats.reshape(B * P, _NSAMPLE, CP),
                          gxyz.reshape(B * P, _NSAMPLE, 4),
                          valid.reshape(B * P, 1),
                          w1f, w1x, b1r, w2t, b2r, B * P, CP)

    new_features = out.reshape(B, P, 128).transpose(0, 2, 1)
    return new_xyz, new_features


# speedup vs baseline: 48.5212x; 1.2443x over previous
import functools

import jax
import jax.numpy as jnp
from jax import lax
from jax.experimental import pallas as pl
from jax.experimental.pallas import tpu as pltpu
from jax.experimental.pallas import tpu_sc as plsc

_RADIUS = 0.1
_NSAMPLE = 32

_NCORES = 2
_NSUB = 16
_NWORKERS = _NCORES * _NSUB


def _sc_ball_query_group(xyz_rows, cent_rows, feats_rows, B, N, P, CP):
    NS = _NSAMPLE
    r2 = jnp.float32(_RADIUS * _RADIUS)
    BLK = 16
    n_blocks = N // (16 * BLK)
    workers_per_batch = _NWORKERS // B
    pc = P // workers_per_batch
    GC = 4
    n_groups = pc // GC

    mesh = plsc.VectorSubcoreMesh(core_axis_name="c", subcore_axis_name="s")

    @functools.partial(
        pl.kernel,
        out_type=[
            jax.ShapeDtypeStruct((B * P * NS, CP), jnp.float32),
            jax.ShapeDtypeStruct((B * P * NS * 4,), jnp.float32),
            jax.ShapeDtypeStruct((B * P,), jnp.float32),
        ],
        mesh=mesh,
        scratch_types=[
            pltpu.VMEM((N,), jnp.float32),
            pltpu.VMEM((N,), jnp.float32),
            pltpu.VMEM((N,), jnp.float32),
            pltpu.VMEM((P,), jnp.float32),
            pltpu.VMEM((P,), jnp.float32),
            pltpu.VMEM((P,), jnp.float32),
            pltpu.VMEM((304,), jnp.int32),
            [pltpu.VMEM((GC * NS,), jnp.int32)] * 2,
            [pltpu.VMEM((GC * NS, CP), jnp.float32)] * 2,
            pltpu.VMEM((pc * NS * 4,), jnp.float32),
            pltpu.VMEM((pc,), jnp.float32),
            [pltpu.SemaphoreType.DMA] * 2,
            [pltpu.SemaphoreType.DMA] * 2,
        ],
        compiler_params=pltpu.CompilerParams(needs_layout_passes=False),
    )
    def sc_kernel(xyz_hbm, cent_hbm, feats_hbm,
                  gfeats_hbm, gxyz_hbm, valid_hbm,
                  xs_v, ys_v, zs_v, cx_v, cy_v, cz_v,
                  cand_v, idxb, fb, gxyz_v, valid_v, sem_g, sem_o):
        wid = lax.axis_index("c") * _NSUB + lax.axis_index("s")
        b = wid // workers_per_batch
        p0 = (wid % workers_per_batch) * pc
        slab = b * P + p0

        pltpu.sync_copy(xyz_hbm.at[b * 3 + 0], xs_v)
        pltpu.sync_copy(xyz_hbm.at[b * 3 + 1], ys_v)
        pltpu.sync_copy(xyz_hbm.at[b * 3 + 2], zs_v)
        pltpu.sync_copy(cent_hbm.at[b * 3 + 0], cx_v)
        pltpu.sync_copy(cent_hbm.at[b * 3 + 1], cy_v)
        pltpu.sync_copy(cent_hbm.at[b * 3 + 2], cz_v)

        lane = lax.iota(jnp.int32, 16)
        zeros16 = jnp.zeros((16,), jnp.int32)
        zerosf = jnp.zeros((16,), jnp.float32)

        def scan_center(j):
            p = p0 + j
            pv = jnp.full((16,), p, jnp.int32)
            cxv = plsc.load_gather(cx_v, [pv])
            cyv = plsc.load_gather(cy_v, [pv])
            czv = plsc.load_gather(cz_v, [pv])

            def scan_cond(carry):
                i, cnt = carry
                return jnp.logical_and(i < n_blocks, cnt < NS)

            def scan_body(carry):
                i, cnt = carry
                base0 = i * (16 * BLK)
                masks, pops = [], []
                for k in range(BLK):
                    base = base0 + k * 16
                    dx = xs_v[pl.ds(base, 16)] - cxv
                    dy = ys_v[pl.ds(base, 16)] - cyv
                    dz = zs_v[pl.ds(base, 16)] - czv
                    d2 = dx * dx + dy * dy + dz * dz
                    masks.append(d2 < r2)
                    pops.append(plsc.all_reduce_population_count(masks[k]))
                sums = [jnp.max(pops[k]) for k in range(BLK)]
                c = cnt
                for k in range(BLK):
                    plsc.store_compressed(cand_v.at[pl.ds(c, 16)],
                                          (base0 + k * 16) + lane,
                                          mask=masks[k])
                    c = c + sums[k]
                return i + 1, c

            _, cnt = lax.while_loop(scan_cond, scan_body,
                                    (jnp.int32(0), jnp.int32(0)))

            has_any = cnt > 0
            c0 = plsc.load_gather(cand_v, [zeros16])
            c0 = jnp.where(has_any, c0, zeros16)
            cntv = jnp.full((16,), cnt, jnp.int32)
            idx0 = jnp.where(lane < cntv, cand_v[pl.ds(0, 16)], c0)
            idx1 = jnp.where(lane + 16 < cntv, cand_v[pl.ds(16, 16)], c0)

            jrow = jnp.full((16,), j * (NS * 4), jnp.int32)
            plsc.store_scatter(gxyz_v, [jrow + lane * 4 + 3], zerosf)
            plsc.store_scatter(gxyz_v, [jrow + (lane + 16) * 4 + 3], zerosf)
            for c, coords in ((0, xs_v), (1, ys_v), (2, zs_v)):
                cc = (cxv, cyv, czv)[c]
                g0 = plsc.load_gather(coords, [idx0]) - cc
                g1 = plsc.load_gather(coords, [idx1]) - cc
                plsc.store_scatter(gxyz_v, [jrow + lane * 4 + c], g0)
                plsc.store_scatter(gxyz_v, [jrow + (lane + 16) * 4 + c], g1)

            vflag = jnp.where(has_any, jnp.float32(1.0), jnp.float32(0.0))
            plsc.store_scatter(valid_v, [jnp.full((16,), j, jnp.int32)],
                               jnp.full((16,), vflag, jnp.float32),
                               mask=lane == 0)
            return idx0, idx1

        def gather_start(g, k):
            pltpu.async_copy(feats_hbm.at[idxb[k]], fb[k], sem_g[k])

        def gather_wait(k):
            pltpu.make_async_copy(feats_hbm.at[idxb[k]], fb[k],
                                  sem_g[k]).wait()

        def out_start(g, k):
            dst = gfeats_hbm.at[pl.ds((slab + g * GC) * NS, GC * NS)]
            pltpu.async_copy(fb[k], dst, sem_o[k])

        def out_wait(g, k):
            dst = gfeats_hbm.at[pl.ds((slab + g * GC) * NS, GC * NS)]
            pltpu.make_async_copy(fb[k], dst, sem_o[k]).wait()

        def pair_body(t, _):
            for k in range(2):
                g = t * 2 + k
                for cloc in range(GC):
                    idx0, idx1 = scan_center(g * GC + cloc)
                    idxb[k][pl.ds(cloc * NS, 16)] = idx0 + b * N
                    idxb[k][pl.ds(cloc * NS + 16, 16)] = idx1 + b * N
                @pl.when(t > 0)
                def _():
                    out_wait(g - 2, k)
                gather_start(g, k)
                @pl.when(g > 0)
                def _():
                    gather_wait(k ^ 1)
                    out_start(g - 1, k ^ 1)
            return 0

        lax.fori_loop(0, n_groups // 2, pair_body, 0)
        gather_wait(1)
        out_start(n_groups - 1, 1)
        out_wait(n_groups - 2, 0)
        out_wait(n_groups - 1, 1)

        pltpu.sync_copy(gxyz_v, gxyz_hbm.at[pl.ds(slab * NS * 4, pc * NS * 4)])
        pltpu.sync_copy(valid_v, valid_hbm.at[pl.ds(slab, pc)])

    return sc_kernel(xyz_rows, cent_rows, feats_rows)


def _tc_mlp_maxpool(gfeats, gxyz, valid, w1f, w1x, b1, w2t, b2, BP, CP):
    NS = _NSAMPLE
    BC = 128

    def body(gf_ref, gx_ref, v_ref, w1f_ref, w1x_ref, b1_ref, w2_ref, b2_ref,
             out_ref):
        gf = gf_ref[...].reshape(BC * NS, CP)
        gx = gx_ref[...].reshape(BC * NS, 4)
        h1 = jnp.dot(gf, w1f_ref[...], preferred_element_type=jnp.float32)
        h1 = h1 + jnp.dot(gx, w1x_ref[...], preferred_element_type=jnp.float32)
        h1 = jnp.maximum(h1 + b1_ref[...], 0.0)
        h2 = jnp.dot(h1, w2_ref[...], preferred_element_type=jnp.float32)
        h2 = jnp.maximum(h2 + b2_ref[...], 0.0)
        h2 = h2.reshape(BC, NS, 128)
        out_ref[...] = jnp.max(h2, axis=1) * v_ref[...]

    return pl.pallas_call(
        body,
        grid=(BP // BC,),
        in_specs=[
            pl.BlockSpec((BC, NS, CP), lambda i: (i, 0, 0)),
            pl.BlockSpec((BC, NS, 4), lambda i: (i, 0, 0)),
            pl.BlockSpec((BC, 1), lambda i: (i, 0)),
            pl.BlockSpec((CP, 64), lambda i: (0, 0)),
            pl.BlockSpec((4, 64), lambda i: (0, 0)),
            pl.BlockSpec((1, 64), lambda i: (0, 0)),
            pl.BlockSpec((64, 128), lambda i: (0, 0)),
            pl.BlockSpec((1, 128), lambda i: (0, 0)),
        ],
        out_specs=pl.BlockSpec((BC, 128), lambda i: (i, 0)),
        out_shape=jax.ShapeDtypeStruct((BP, 128), jnp.float32),
    )(gfeats, gxyz, valid, w1f, w1x, b1, w2t, b2)


def kernel(xyz, features, new_xyz, W1, b1, W2, b2):
    B, N, _ = xyz.shape
    P = new_xyz.shape[1]
    C = features.shape[1]

    CP = 128
    xyz_rows = xyz.transpose(0, 2, 1).reshape(B * 3, N)
    cent_rows = new_xyz.transpose(0, 2, 1).reshape(B * 3, P)
    feats_rows = jnp.pad(features.transpose(0, 2, 1).reshape(B * N, C),
                         ((0, 0), (0, CP - C)))

    gfeats, gxyz, valid = _sc_ball_query_group(
        xyz_rows, cent_rows, feats_rows, B, N, P, CP)

    w1f = jnp.pad(W1[:, 3:].T, ((0, CP - C), (0, 0)))
    w1x = jnp.pad(W1[:, :3], ((0, 0), (0, 1))).T
    b1r = b1.reshape(1, 64)
    w2t = W2.T
    b2r = b2.reshape(1, 128)

    out = _tc_mlp_maxpool(gfeats.reshape(B * P, _NSAMPLE, CP),
                          gxyz.reshape(B * P, _NSAMPLE, 4),
                          valid.reshape(B * P, 1),
                          w1f, w1x, b1r, w2t, b2r, B * P, CP)

    new_features = out.reshape(B, P, 128).transpose(0, 2, 1)
    return new_xyz, new_features
